# Initial kernel scaffold; baseline (speedup 1.0000x reference)
import functools

import jax
import jax.numpy as jnp
from jax import lax
from jax.experimental import pallas as pl
from jax.experimental.pallas import tpu as pltpu
from jax.experimental.pallas import tpu_sc as plsc

N = 100000
D = 128
V = 10
NC, NS = 2, 16
NW = NC * NS
CHUNK = 128
NCHUNKS = (N + CHUNK - 1) // CHUNK


@functools.partial(
    pl.kernel,
    out_type=jax.ShapeDtypeStruct((N, D), jnp.float32),
    mesh=plsc.VectorSubcoreMesh(core_axis_name="c", subcore_axis_name="s"),
    scratch_types=[
        pltpu.VMEM_SHARED((V, D), jnp.float32),
        pltpu.VMEM((CHUNK,), jnp.int32),
        pltpu.VMEM((CHUNK, D), jnp.float32),
        pltpu.SemaphoreType.DMA,
    ],
)
def _lookup(idx_hbm, tab_hbm, out_hbm, tab_sh, idx_v, rows_v, sem):
    cid = lax.axis_index("c")
    sid = lax.axis_index("s")
    wid = sid * NC + cid

    @pl.when(sid == 0)
    def _():
        pltpu.sync_copy(tab_hbm, tab_sh)

    plsc.subcore_barrier()

    nmine = (NCHUNKS - wid + NW - 1) // NW

    def body(i, carry):
        c = wid + i * NW
        start = jnp.minimum(c * CHUNK, N - CHUNK)
        pltpu.sync_copy(idx_hbm.at[pl.ds(start, CHUNK)], idx_v)
        pltpu.async_copy(tab_sh.at[idx_v], rows_v, sem).wait()
        pltpu.sync_copy(rows_v, out_hbm.at[pl.ds(start, CHUNK)])
        return carry

    lax.fori_loop(0, nmine, body, 0)


def kernel(atomic_num, embedding):
    idx = atomic_num.astype(jnp.int32)
    return _lookup(idx, embedding)


# speedup vs baseline: 3.4827x; 3.4827x over previous
import functools

import jax
import jax.numpy as jnp
from jax import lax
from jax.experimental import pallas as pl
from jax.experimental.pallas import tpu as pltpu
from jax.experimental.pallas import tpu_sc as plsc

N = 100000
D = 128
V = 10
NC, NS = 2, 16
NW = NC * NS
CHUNK = 128
NCHUNKS = (N + CHUNK - 1) // CHUNK


@functools.lru_cache(maxsize=1)
def _build():
    @functools.partial(
        pl.kernel,
        out_type=jax.ShapeDtypeStruct((N, D), jnp.float32),
        mesh=plsc.VectorSubcoreMesh(core_axis_name="c", subcore_axis_name="s"),
        scratch_types=[
            pltpu.VMEM_SHARED((V, D), jnp.float32),
            pltpu.VMEM((CHUNK,), jnp.int32),
            pltpu.VMEM((CHUNK, D), jnp.float32),
            pltpu.SemaphoreType.DMA,
        ],
    )
    def _lookup(idx_hbm, tab_hbm, out_hbm, tab_sh, idx_v, rows_v, sem):
        cid = lax.axis_index("c")
        sid = lax.axis_index("s")
        wid = sid * NC + cid

        @pl.when(sid == 0)
        def _():
            pltpu.sync_copy(tab_hbm, tab_sh)

        plsc.subcore_barrier()

        nmine = (NCHUNKS - wid + NW - 1) // NW

        def body(i, carry):
            c = wid + i * NW
            start = jnp.minimum(c * CHUNK, N - CHUNK)
            pltpu.sync_copy(idx_hbm.at[pl.ds(start, CHUNK)], idx_v)
            pltpu.async_copy(tab_sh.at[idx_v], rows_v, sem).wait()
            pltpu.sync_copy(rows_v, out_hbm.at[pl.ds(start, CHUNK)])
            return carry

        lax.fori_loop(0, nmine, body, 0)

    return _lookup


def kernel(atomic_num, embedding):
    idx = atomic_num.astype(jnp.int32)
    return _build()(idx, embedding)


# speedup vs baseline: 5.4210x; 1.5566x over previous
import functools

import jax
import jax.numpy as jnp
from jax import lax
from jax.experimental import pallas as pl
from jax.experimental.pallas import tpu as pltpu
from jax.experimental.pallas import tpu_sc as plsc

N = 100000
D = 128
V = 10
NC, NS = 2, 16
NW = NC * NS
CHUNK = 128
NCHUNKS = (N + CHUNK - 1) // CHUNK
T = (NCHUNKS + NW - 1) // NW
SPAN = T * CHUNK


@functools.lru_cache(maxsize=1)
def _build():
    @functools.partial(
        pl.kernel,
        out_type=jax.ShapeDtypeStruct((N, D), jnp.float32),
        mesh=plsc.VectorSubcoreMesh(core_axis_name="c", subcore_axis_name="s"),
        scratch_types=[
            pltpu.VMEM_SHARED((V, D), jnp.float32),
            pltpu.VMEM((SPAN,), jnp.int32),
            pltpu.VMEM((CHUNK, D), jnp.float32),
            pltpu.VMEM((CHUNK, D), jnp.float32),
            pltpu.SemaphoreType.DMA,
            pltpu.SemaphoreType.DMA,
            pltpu.SemaphoreType.DMA,
            pltpu.SemaphoreType.DMA,
        ],
    )
    def _lookup(idx_hbm, tab_hbm, out_hbm, tab_sh, idx_all, rows0, rows1,
                g0, g1, w0, w1):
        cid = lax.axis_index("c")
        sid = lax.axis_index("s")
        wid = sid * NC + cid

        @pl.when(sid == 0)
        def _():
            pltpu.sync_copy(tab_hbm, tab_sh)

        plsc.subcore_barrier()

        span = jnp.minimum((wid * NCHUNKS) // NW * CHUNK, N - SPAN)
        pltpu.sync_copy(idx_hbm.at[pl.ds(span, SPAN)], idx_all)

        rows = (rows0, rows1)
        gsem = (g0, g1)
        wsem = (w0, w1)

        def start_gather(i, b):
            return pltpu.async_copy(
                tab_sh.at[idx_all.at[pl.ds(i * CHUNK, CHUNK)]], rows[b], gsem[b]
            )

        gd = [start_gather(0, 0), None]
        wd = [None, None]
        for i in range(T):
            b = i & 1
            nb = 1 - b
            if i + 1 < T:
                if wd[nb] is not None:
                    wd[nb].wait()
                gd[nb] = start_gather(i + 1, nb)
            gd[b].wait()
            wd[b] = pltpu.async_copy(
                rows[b], out_hbm.at[pl.ds(span + i * CHUNK, CHUNK)], wsem[b]
            )
        for d in wd:
            if d is not None:
                d.wait()

    return _lookup


def kernel(atomic_num, embedding):
    idx = atomic_num.astype(jnp.int32)
    return _build()(idx, embedding)
